# Initial kernel scaffold; baseline (speedup 1.0000x reference)
import jax
import jax.numpy as jnp
from jax.experimental import pallas as pl


def kernel(x, edge_index, W, b):
    raise NotImplementedError("write your pallas kernel here")


# speedup vs baseline: 14.5687x; 14.5687x over previous
import functools

import jax
import jax.numpy as jnp
from jax import lax
from jax.experimental import pallas as pl
from jax.experimental.pallas import tpu as pltpu
from jax.experimental.pallas import tpu_sc as plsc

NC = 2
NS = 16
NW = NC * NS
CH = 128
DEGW = 16


def _mesh():
    return plsc.VectorSubcoreMesh(core_axis_name="c", subcore_axis_name="s")


def _sc_degree(dstp, n, npad, t):
    def body(dst_hbm, deg_out, deg_sh, ones_v, zero_v, dst_v):
        cid = lax.axis_index("c")
        sid = lax.axis_index("s")
        wid = sid * NC + cid

        def fill(i, _):
            ones_v[i, :] = jnp.full((DEGW,), 1.0, jnp.float32)
            zero_v[i, :] = jnp.zeros((DEGW,), jnp.float32)
            return _

        lax.fori_loop(0, CH, fill, 0)

        def zinit(j, _):
            pltpu.sync_copy(zero_v, deg_sh.at[pl.ds(sid * (npad // NS) + j * CH, CH)])
            return _

        lax.fori_loop(0, npad // NS // CH, zinit, 0)
        plsc.subcore_barrier()

        base = wid * t * CH

        def step(i, _):
            pltpu.sync_copy(dst_hbm.at[pl.ds(base + i * CH, CH)], dst_v)
            pltpu.sync_copy(ones_v, deg_sh.at[dst_v], add=True)
            return _

        lax.fori_loop(0, t, step, 0)
        plsc.subcore_barrier()

        cp = npad // NS
        pltpu.sync_copy(deg_sh.at[pl.ds(sid * cp, cp)],
                        deg_out.at[cid, pl.ds(sid * cp, cp)])

    return pl.kernel(
        body,
        out_type=jax.ShapeDtypeStruct((NC, npad, DEGW), jnp.float32),
        mesh=_mesh(),
        scratch_types=[
            pltpu.VMEM_SHARED((npad, DEGW), jnp.float32),
            pltpu.VMEM((CH, DEGW), jnp.float32),
            pltpu.VMEM((CH, DEGW), jnp.float32),
            pltpu.VMEM((CH,), jnp.int32),
        ],
    )(dstp)


def _sc_scatter(hp, srcp, dstp, n, npad, t, d):
    def body(hp_hbm, src_hbm, dst_hbm, acc_out, acc_sh, rows_v, src_v, dst_v, sem):
        cid = lax.axis_index("c")
        sid = lax.axis_index("s")
        wid = sid * NC + cid

        def zrow(i, _):
            rows_v[i // 8, pl.ds((i % 8) * 16, 16)] = jnp.zeros((16,), jnp.float32)
            return _

        lax.fori_loop(0, CH * (d // 16), zrow, 0)

        def zinit(j, _):
            pltpu.sync_copy(rows_v, acc_sh.at[pl.ds(sid * (npad // NS) + j * CH, CH)])
            return _

        lax.fori_loop(0, npad // NS // CH, zinit, 0)
        plsc.subcore_barrier()

        base = wid * t * CH

        def step(i, _):
            pltpu.sync_copy(src_hbm.at[pl.ds(base + i * CH, CH)], src_v)
            pltpu.sync_copy(dst_hbm.at[pl.ds(base + i * CH, CH)], dst_v)
            pltpu.async_copy(hp_hbm.at[src_v], rows_v, sem).wait()
            pltpu.sync_copy(rows_v, acc_sh.at[dst_v], add=True)
            return _

        lax.fori_loop(0, t, step, 0)
        plsc.subcore_barrier()

        cp = npad // NS
        pltpu.sync_copy(acc_sh.at[pl.ds(sid * cp, cp)],
                        acc_out.at[cid, pl.ds(sid * cp, cp)])

    return pl.kernel(
        body,
        out_type=jax.ShapeDtypeStruct((NC, npad, d), jnp.float32),
        mesh=_mesh(),
        scratch_types=[
            pltpu.VMEM_SHARED((npad, d), jnp.float32),
            pltpu.VMEM((CH, d), jnp.float32),
            pltpu.VMEM((CH,), jnp.int32),
            pltpu.VMEM((CH,), jnp.int32),
            pltpu.SemaphoreType.DMA,
        ],
    )(hp, srcp, dstp)


def _tc_mm_scale(x, w, deg_parts, bn):
    n, d_in = x.shape
    d_out = w.shape[1]

    def body(x_ref, w_ref, dp_ref, o_ref):
        h = jnp.dot(x_ref[...], w_ref[...], preferred_element_type=jnp.float32)
        deg = dp_ref[0, :, 0:1] + dp_ref[1, :, 0:1] + 1.0
        o_ref[...] = h * lax.rsqrt(deg)

    return pl.pallas_call(
        body,
        grid=(n // bn,),
        in_specs=[
            pl.BlockSpec((bn, d_in), lambda i: (i, 0)),
            pl.BlockSpec((d_in, d_out), lambda i: (0, 0)),
            pl.BlockSpec((NC, bn, DEGW), lambda i: (0, i, 0)),
        ],
        out_specs=pl.BlockSpec((bn, d_out), lambda i: (i, 0)),
        out_shape=jax.ShapeDtypeStruct((n, d_out), jnp.float32),
    )(x, w, deg_parts)


def _tc_combine(acc_parts, hp, deg_parts, b2, bn):
    n, d = hp.shape

    def body(ap_ref, hp_ref, dp_ref, b_ref, o_ref):
        deg = dp_ref[0, :, 0:1] + dp_ref[1, :, 0:1] + 1.0
        s = ap_ref[0] + ap_ref[1] + hp_ref[...]
        o_ref[...] = s * lax.rsqrt(deg) + b_ref[...]

    return pl.pallas_call(
        body,
        grid=(n // bn,),
        in_specs=[
            pl.BlockSpec((NC, bn, d), lambda i: (0, i, 0)),
            pl.BlockSpec((bn, d), lambda i: (i, 0)),
            pl.BlockSpec((NC, bn, DEGW), lambda i: (0, i, 0)),
            pl.BlockSpec((1, d), lambda i: (0, 0)),
        ],
        out_specs=pl.BlockSpec((bn, d), lambda i: (i, 0)),
        out_shape=jax.ShapeDtypeStruct((n, d), jnp.float32),
    )(acc_parts, hp, deg_parts, b2)


@jax.jit
def kernel(x, edge_index, W, b):
    n, d_in = x.shape
    d_out = W.shape[1]
    e = edge_index.shape[1]

    t = -(-e // (NW * CH))
    ep = t * NW * CH
    src = edge_index[0]
    dst = edge_index[1]
    srcp = jnp.concatenate([src, jnp.zeros((ep - e,), jnp.int32)])
    dstp = jnp.concatenate([dst, jnp.full((ep - e,), n, jnp.int32)])

    npad = NS * CH * (-(-(n + 1) // (NS * CH)))

    deg_parts = _sc_degree(dstp, n, npad, t)
    hp = _tc_mm_scale(x, W, deg_parts, bn=1000)
    acc_parts = _sc_scatter(hp, srcp, dstp, n, npad, t, d_out)
    return _tc_combine(acc_parts, hp, deg_parts, b.reshape(1, d_out), bn=1000)
